# Initial kernel scaffold; baseline (speedup 1.0000x reference)
import functools

import jax
import jax.numpy as jnp
from jax import lax
from jax.experimental import pallas as pl
from jax.experimental.pallas import tpu as pltpu
from jax.experimental.pallas import tpu_sc as plsc

NC = 2
NS = 16
L = 16
NW = NC * NS


@functools.lru_cache(maxsize=None)
def _make_lookup(n, chunk):
    n_w = n // NW
    nchunk = n_w // chunk
    iters = chunk // L

    mesh = plsc.VectorSubcoreMesh(core_axis_name="c", subcore_axis_name="s")

    @functools.partial(
        pl.kernel,
        out_type=jax.ShapeDtypeStruct((n * 3,), jnp.float32),
        mesh=mesh,
        scratch_types=[
            pltpu.VMEM((3 * L,), jnp.float32),
            pltpu.VMEM((chunk,), jnp.int32),
            pltpu.VMEM((chunk * 3,), jnp.float32),
        ],
    )
    def lookup(table_hbm, idx_hbm, out_hbm, table_v, idx_v, out_v):
        wid = lax.axis_index("s") * NC + lax.axis_index("c")
        base = wid * n_w
        pltpu.sync_copy(table_hbm, table_v)
        pos = lax.iota(jnp.int32, L) * 3

        def chunk_body(ci, carry):
            off = base + ci * chunk
            pltpu.sync_copy(idx_hbm.at[pl.ds(off, chunk)], idx_v)

            def inner(i, c):
                idx = idx_v[pl.ds(i * L, L)]
                o = out_v.at[pl.ds(i * (3 * L), 3 * L)]
                for d in range(3):
                    v = plsc.load_gather(table_v, [idx + d * L])
                    plsc.store_scatter(o, [pos + d], v)
                return c

            lax.fori_loop(0, iters, inner, 0, unroll=4)
            pltpu.sync_copy(out_v, out_hbm.at[pl.ds(off * 3, chunk * 3)])
            return carry

        lax.fori_loop(0, nchunk, chunk_body, 0)

    return lookup


def kernel(arg, weight):
    rows, cols = arg.shape
    n = rows * cols
    idx = arg.reshape(-1).astype(jnp.int32)
    table = jnp.zeros((3, L), jnp.float32).at[:, : weight.shape[0]].set(weight.T)
    out = _make_lookup(n, 10240)(table.reshape(-1), idx)
    return out.reshape(rows, cols, 3)


# speedup vs baseline: 5.5767x; 5.5767x over previous
import functools

import jax
import jax.numpy as jnp
from jax import lax
from jax.experimental import pallas as pl
from jax.experimental.pallas import tpu as pltpu
from jax.experimental.pallas import tpu_sc as plsc

NC = 2
NS = 16
L = 16
NW = NC * NS


@functools.lru_cache(maxsize=None)
def _make_lookup(n, chunk):
    n_w = n // NW
    nchunk = n_w // chunk
    iters = chunk // L

    mesh = plsc.VectorSubcoreMesh(core_axis_name="c", subcore_axis_name="s")

    @functools.partial(
        pl.kernel,
        out_type=jax.ShapeDtypeStruct((n * 3,), jnp.float32),
        mesh=mesh,
        scratch_types=[
            pltpu.VMEM((3 * L,), jnp.float32),
            pltpu.VMEM((chunk,), jnp.int32),
            pltpu.VMEM((chunk * 3,), jnp.float32),
        ],
        compiler_params=pltpu.CompilerParams(needs_layout_passes=False),
    )
    def lookup(table_hbm, idx_hbm, out_hbm, table_v, idx_v, out_v):
        wid = lax.axis_index("s") * NC + lax.axis_index("c")
        base = wid * n_w
        pltpu.sync_copy(table_hbm, table_v)
        pos = lax.iota(jnp.int32, L) * 3

        def chunk_body(ci, carry):
            off = base + ci * chunk
            pltpu.sync_copy(idx_hbm.at[pl.ds(off, chunk)], idx_v)

            def inner(i, c):
                idx = idx_v[pl.ds(i * L, L)]
                o = out_v.at[pl.ds(i * (3 * L), 3 * L)]
                for d in range(3):
                    v = plsc.load_gather(table_v, [idx + d * L])
                    plsc.store_scatter(o, [pos + d], v)
                return c

            lax.fori_loop(0, iters, inner, 0, unroll=4)
            pltpu.sync_copy(out_v, out_hbm.at[pl.ds(off * 3, chunk * 3)])
            return carry

        lax.fori_loop(0, nchunk, chunk_body, 0)

    return lookup


def kernel(arg, weight):
    rows, cols = arg.shape
    n = rows * cols
    idx = arg.reshape(-1).astype(jnp.int32)
    table = jnp.zeros((3, L), jnp.float32).at[:, : weight.shape[0]].set(weight.T)
    out = _make_lookup(n, 10240)(table.reshape(-1), idx)
    return out.reshape(rows, cols, 3)
